# baseline (device time: 19734 ns/iter reference)
import jax
import jax.numpy as jnp
from jax import lax
from jax.experimental import pallas as pl
from jax.experimental.pallas import tpu as pltpu

Z = 2
CZ = 4


def kernel(x, dy):
    k, d = x.shape
    _, f = dy.shape
    d_half = d // Z
    quarter = d_half // 4
    chunk = quarter // CZ
    fh = f // 2

    def body(x_hbm, dy_hbm, out_hbm,
             xv, dyv, outv, blk16, zrecv, xrecv, yrecv, drecv,
             in_s, ob_s,
             zs_s, zr_s, fxs_s, fxr_s, fys_s, fyr_s, g_s, gr_s):
        mx = lax.axis_index("x")
        my = lax.axis_index("y")
        mz = lax.axis_index("z")
        zpeer = (mx, my, 1 - mz)
        xpeer = (1 - mx, my, mz)
        ypeer = (mx, 1 - my, mz)
        q_me = 2 * mx + my
        q_x = 2 * (1 - mx) + my
        q_y = 2 * mx + (1 - my)
        q_d = 2 * (1 - mx) + (1 - my)

        barrier = pltpu.get_barrier_semaphore()
        for nbr in (zpeer, xpeer, ypeer):
            pl.semaphore_signal(
                barrier, inc=1, device_id=nbr,
                device_id_type=pl.DeviceIdType.MESH,
            )

        cp_x = pltpu.make_async_copy(x_hbm, xv, in_s.at[0])
        cp_d0 = pltpu.make_async_copy(
            dy_hbm.at[:, pl.ds(0, fh)], dyv.at[:, pl.ds(0, fh)], in_s.at[1]
        )
        cp_d1 = pltpu.make_async_copy(
            dy_hbm.at[:, pl.ds(fh, fh)], dyv.at[:, pl.ds(fh, fh)], in_s.at[2]
        )
        cp_x.start()
        cp_d0.start()
        cp_d1.start()

        pcol = (1 - mz) * d_half + mx * (2 * quarter)
        cp_x.wait()
        cp_d0.wait()
        blk_f0 = lax.dot_general(
            xv[:, pl.ds(pcol, 2 * quarter)],
            dyv[:, pl.ds(0, fh)],
            (((0,), (0,)), ((), ())),
            preferred_element_type=jnp.float32,
        )
        blk16[:, :, pl.ds(0, fh)] = (
            blk_f0.astype(jnp.bfloat16).reshape(2 * CZ, chunk, fh)
        )
        cp_d1.wait()
        blk_f1 = lax.dot_general(
            xv[:, pl.ds(pcol, 2 * quarter)],
            dyv[:, pl.ds(fh, fh)],
            (((0,), (0,)), ((), ())),
            preferred_element_type=jnp.float32,
        )
        blk16[:, :, pl.ds(fh, fh)] = (
            blk_f1.astype(jnp.bfloat16).reshape(2 * CZ, chunk, fh)
        )

        pl.semaphore_wait(barrier, 3)

        zs = []
        for c in range(CZ):
            r = pltpu.make_async_remote_copy(
                src_ref=blk16.at[my * CZ + c],
                dst_ref=zrecv.at[c],
                send_sem=zs_s.at[c],
                recv_sem=zr_s.at[c],
                device_id=zpeer,
                device_id_type=pl.DeviceIdType.MESH,
            )
            r.start()
            zs.append(r)

        mcol = mz * d_half
        outv[:, :] = lax.dot_general(
            xv[:, pl.ds(mcol, d_half)],
            dyv[:, :],
            (((0,), (0,)), ((), ())),
            preferred_element_type=jnp.float32,
        )

        fx, fy = [], []
        for c in range(CZ):
            zs[c].wait_recv()
            for lst, dst, ss, rs, nbr in (
                (fx, xrecv, fxs_s, fxr_s, xpeer),
                (fy, yrecv, fys_s, fyr_s, ypeer),
            ):
                r = pltpu.make_async_remote_copy(
                    src_ref=zrecv.at[c],
                    dst_ref=dst.at[c],
                    send_sem=ss.at[c],
                    recv_sem=rs.at[c],
                    device_id=nbr,
                    device_id_type=pl.DeviceIdType.MESH,
                )
                r.start()
                lst.append(r)

        obs = []

        def acc_and_store(q, buf):
            outv[pl.ds(q * quarter, quarter), :] = (
                outv[pl.ds(q * quarter, quarter), :]
                + buf[...].reshape(quarter, f).astype(jnp.float32)
            )
            ob = pltpu.make_async_copy(
                outv.at[pl.ds(q * quarter, quarter), :],
                out_hbm.at[pl.ds(q * quarter, quarter), :],
                ob_s.at[len(obs)],
            )
            ob.start()
            obs.append(ob)

        acc_and_store(q_me, zrecv)

        g = []
        for c in range(CZ):
            src, nbr = (xrecv, ypeer) if c < CZ // 2 else (yrecv, xpeer)
            waiter = fx[c] if c < CZ // 2 else fy[c]
            waiter.wait_recv()
            r = pltpu.make_async_remote_copy(
                src_ref=src.at[c],
                dst_ref=drecv.at[c],
                send_sem=g_s.at[c],
                recv_sem=gr_s.at[c],
                device_id=nbr,
                device_id_type=pl.DeviceIdType.MESH,
            )
            r.start()
            g.append(r)

        for c in range(CZ // 2, CZ):
            fx[c].wait_recv()
        acc_and_store(q_x, xrecv)
        for c in range(CZ // 2):
            fy[c].wait_recv()
        acc_and_store(q_y, yrecv)

        for r in g:
            r.wait_recv()
        acc_and_store(q_d, drecv)

        for ob in obs:
            ob.wait()
        for r in zs + fx + fy + g:
            r.wait_send()

    return pl.pallas_call(
        body,
        out_shape=jax.ShapeDtypeStruct((d_half, f), jnp.float32),
        in_specs=[
            pl.BlockSpec(memory_space=pltpu.ANY),
            pl.BlockSpec(memory_space=pltpu.ANY),
        ],
        out_specs=pl.BlockSpec(memory_space=pltpu.ANY),
        scratch_shapes=[
            pltpu.VMEM((k, d), jnp.float32),
            pltpu.VMEM((k, f), jnp.float32),
            pltpu.VMEM((d_half, f), jnp.float32),
            pltpu.VMEM((2 * CZ, chunk, f), jnp.bfloat16),
            pltpu.VMEM((CZ, chunk, f), jnp.bfloat16),
            pltpu.VMEM((CZ, chunk, f), jnp.bfloat16),
            pltpu.VMEM((CZ, chunk, f), jnp.bfloat16),
            pltpu.VMEM((CZ, chunk, f), jnp.bfloat16),
            pltpu.SemaphoreType.DMA((3,)),
            pltpu.SemaphoreType.DMA((4,)),
            pltpu.SemaphoreType.DMA((CZ,)),
            pltpu.SemaphoreType.DMA((CZ,)),
            pltpu.SemaphoreType.DMA((CZ,)),
            pltpu.SemaphoreType.DMA((CZ,)),
            pltpu.SemaphoreType.DMA((CZ,)),
            pltpu.SemaphoreType.DMA((CZ,)),
            pltpu.SemaphoreType.DMA((CZ,)),
            pltpu.SemaphoreType.DMA((CZ,)),
        ],
        compiler_params=pltpu.CompilerParams(collective_id=0),
    )(x, dy)
